# baseline (speedup 1.0000x reference)
import functools

import jax
import jax.numpy as jnp
from jax.experimental import pallas as pl

_ALPHA = 0.2
_NEG = -9e15
_BLK = 256


def _proj_body(h, x_ref, wn_ref, an_ref, wh_ref, f1_ref, f2_ref):
    wh = jnp.dot(x_ref[...], wn_ref[...], preferred_element_type=jnp.float32)
    wh_ref[...] = wh
    f1_ref[...] = jnp.dot(wh, an_ref[:h, :], preferred_element_type=jnp.float32)
    f2_ref[...] = jnp.dot(wh, an_ref[h:, :], preferred_element_type=jnp.float32)


def _proj(x, wn, an):
    r = x.shape[0]
    h = wn.shape[1]
    return pl.pallas_call(
        functools.partial(_proj_body, h),
        out_shape=(
            jax.ShapeDtypeStruct((r, h), jnp.float32),
            jax.ShapeDtypeStruct((r, 1), jnp.float32),
            jax.ShapeDtypeStruct((r, 1), jnp.float32),
        ),
    )(x, wn, an)


def _att_body(final, mask_ref, f1_ref, f2_ref, w_ref, o_ref):
    s = f1_ref[...] + f2_ref[...]
    s = jnp.where(s >= 0, s, _ALPHA * s)
    s = jnp.where(mask_ref[...].astype(jnp.float32) > 0.0, s, _NEG)
    m = jnp.max(s, axis=1, keepdims=True)
    p = jnp.exp(s - m)
    den = jnp.sum(p, axis=1, keepdims=True)
    h = jnp.dot(p, w_ref[...], preferred_element_type=jnp.float32) / den
    if final:
        h = jnp.where(h > 0, h, jnp.exp(h) - 1.0)
        h = h - jnp.max(h, axis=1, keepdims=True)
        h = h - jnp.log(jnp.sum(jnp.exp(h), axis=1, keepdims=True))
    o_ref[...] = h


def _att(mask8, f1, f2row, w, final=False):
    r, c = mask8.shape
    h = w.shape[1]
    return pl.pallas_call(
        functools.partial(_att_body, final),
        grid=(r // _BLK,),
        in_specs=[
            pl.BlockSpec((_BLK, c), lambda i: (i, 0)),
            pl.BlockSpec((_BLK, 1), lambda i: (i, 0)),
            pl.BlockSpec((1, c), lambda i: (0, 0)),
            pl.BlockSpec((c, h), lambda i: (0, 0)),
        ],
        out_specs=pl.BlockSpec((_BLK, h), lambda i: (i, 0)),
        out_shape=jax.ShapeDtypeStruct((r, h), jnp.float32),
    )(mask8, f1, f2row, w)


def _cat_body(k, ne_ref, h_ref, ep_ref, o_ref):
    pooled = jnp.dot(ne_ref[...].astype(jnp.float32), ep_ref[...],
                     preferred_element_type=jnp.float32)
    hh = h_ref[...]
    o_ref[:, :k] = jnp.where(hh > 0, hh, jnp.exp(hh) - 1.0)
    o_ref[:, k:] = jnp.where(pooled > 0, pooled, jnp.exp(pooled) - 1.0)


def _cat(ne8, hp, ep):
    r, c = ne8.shape
    k = hp.shape[1]
    return pl.pallas_call(
        functools.partial(_cat_body, k),
        grid=(r // _BLK,),
        in_specs=[
            pl.BlockSpec((_BLK, c), lambda i: (i, 0)),
            pl.BlockSpec((_BLK, k), lambda i: (i, 0)),
            pl.BlockSpec((c, k), lambda i: (0, 0)),
        ],
        out_specs=pl.BlockSpec((_BLK, 2 * k), lambda i: (i, 0)),
        out_shape=jax.ShapeDtypeStruct((r, 2 * k), jnp.float32),
    )(ne8, hp, ep)


def _pool_mlp_body(a_ref, h_ref, w1_ref, b1_ref, w2_ref, b2_ref,
                   z_ref, s_ref, ss_ref):
    pooled = jnp.dot(a_ref[...].astype(jnp.float32), h_ref[...],
                     preferred_element_type=jnp.float32)
    t = jnp.maximum(
        jnp.dot(pooled, w1_ref[...], preferred_element_type=jnp.float32)
        + b1_ref[...], 0.0)
    z = jnp.dot(t, w2_ref[...], preferred_element_type=jnp.float32) + b2_ref[...]
    z_ref[...] = z

    @pl.when(pl.program_id(0) == 0)
    def _init():
        s_ref[...] = jnp.zeros_like(s_ref)
        ss_ref[...] = jnp.zeros_like(ss_ref)

    s_ref[...] += jnp.sum(z, axis=0, keepdims=True)
    ss_ref[...] += jnp.sum(z * z, axis=0, keepdims=True)


def _pool_mlp(a8, h, mp):
    r, c = a8.shape
    f = h.shape[1]
    k1 = mp['W1'].shape[1]
    k2 = mp['W2'].shape[1]
    return pl.pallas_call(
        _pool_mlp_body,
        grid=(r // _BLK,),
        in_specs=[
            pl.BlockSpec((_BLK, c), lambda i: (i, 0)),
            pl.BlockSpec((c, f), lambda i: (0, 0)),
            pl.BlockSpec((f, k1), lambda i: (0, 0)),
            pl.BlockSpec((1, k1), lambda i: (0, 0)),
            pl.BlockSpec((k1, k2), lambda i: (0, 0)),
            pl.BlockSpec((1, k2), lambda i: (0, 0)),
        ],
        out_specs=(
            pl.BlockSpec((_BLK, k2), lambda i: (i, 0)),
            pl.BlockSpec((1, k2), lambda i: (0, 0)),
            pl.BlockSpec((1, k2), lambda i: (0, 0)),
        ),
        out_shape=(
            jax.ShapeDtypeStruct((r, k2), jnp.float32),
            jax.ShapeDtypeStruct((1, k2), jnp.float32),
            jax.ShapeDtypeStruct((1, k2), jnp.float32),
        ),
    )(a8, h, mp['W1'], mp['b1'].reshape(1, -1), mp['W2'],
      mp['b2'].reshape(1, -1))


def _bn_body(n, z_ref, s_ref, ss_ref, g_ref, b_ref, o_ref):
    mu = s_ref[...] / n
    var = ss_ref[...] / n - mu * mu
    inv = 1.0 / jnp.sqrt(var + 1e-5)
    o_ref[...] = jnp.maximum((z_ref[...] - mu) * inv * g_ref[...] + b_ref[...],
                             0.0)


def _bn_relu(z, s, ss, bp):
    r, k = z.shape
    return pl.pallas_call(
        functools.partial(_bn_body, float(r)),
        out_shape=jax.ShapeDtypeStruct((r, k), jnp.float32),
    )(z, s, ss, bp['gamma'].reshape(1, -1), bp['beta'].reshape(1, -1))


def _gat_layer(x, e_x, adj8, eadj8, nea8, neaT8, p, concat, final=False):
    wh, f1, f2 = _proj(x, p['Wn'], p['an'])
    we, g1, g2 = _proj(e_x, p['We'], p['ae'])
    hp = _att(adj8, f1, f2.reshape(1, -1), wh, final=final)
    ep = _att(eadj8, g1, g2.reshape(1, -1), we, final=final)
    if not concat:
        return hp, ep
    xo = _cat(nea8, hp, ep)
    eo = _cat(neaT8, ep, hp)
    return xo, eo


def _next(h, a8, mp, bp):
    z, s, ss = _pool_mlp(a8, h, mp)
    return _bn_relu(z, s, ss, bp)


def kernel(x, e_x, adj, e_adj, n_e_adj, params):
    adj8 = adj.astype(jnp.int8)
    eadj8 = e_adj.astype(jnp.int8)
    nea8 = n_e_adj.astype(jnp.int8)
    neaT8 = n_e_adj.T.astype(jnp.int8)
    x1, e1 = _gat_layer(x, e_x, adj8, eadj8, nea8, neaT8,
                        params['in_att'], True)
    x2, e2 = _gat_layer(x1, e1, adj8, eadj8, nea8, neaT8,
                        params['att0'], True)
    hn, he = x2, e2
    for mk, bk in (('mlp0', 'bn0'), ('mlp1', 'bn1')):
        hn = _next(hn, adj8, params[mk], params[bk])
        he = _next(he, eadj8, params[mk], params[bk])
    fx, fe = _gat_layer(hn, he, adj8, eadj8, nea8, neaT8,
                        params['out_att'], False, final=True)
    return fx, fe


# speedup vs baseline: 1.4435x; 1.4435x over previous
import functools

import jax
import jax.numpy as jnp
from jax.experimental import pallas as pl

_ALPHA = 0.2
_BLK = 256
_EPS = 1e-5


def _abcd(wh, an, h):
    f1 = jnp.dot(wh, an[:h, :], preferred_element_type=jnp.float32)
    f2 = jnp.dot(wh, an[h:, :], preferred_element_type=jnp.float32)
    mf2 = jnp.max(f2)
    t = f1 + mf2
    m = jnp.maximum(t, _ALPHA * t)
    a = jnp.exp(t - m)
    c = jnp.exp(_ALPHA * t - m)
    b = jnp.exp(f2 - mf2)
    d = jnp.exp(_ALPHA * (f2 - mf2))
    return a, c, b, d


def _elu(x):
    return jnp.where(x > 0, x, jnp.exp(x) - 1.0)


def _proj_side(x, wn, an, h, wa_ref, wm_ref, a_ref, c_ref, b_ref, d_ref):
    wh = jnp.dot(x, wn, preferred_element_type=jnp.float32)
    wa_ref[:, :h] = wh
    wa_ref[:, h:] = jnp.ones_like(wa_ref[:, h:])
    wm_ref[...] = jnp.mean(wh, axis=0, keepdims=True)
    a, c, b, d = _abcd(wh, an, h)
    a_ref[...] = a
    c_ref[...] = c
    b_ref[...] = b
    d_ref[...] = d


def _proj_body(hn, he, xn_ref, wn_ref, an_ref, xe_ref, we_ref, ae_ref,
               wan_ref, wmn_ref, an1_ref, cn_ref, bn1_ref, dn_ref,
               wae_ref, wme_ref, ae1_ref, ce_ref, be_ref, de_ref):
    _proj_side(xn_ref[...], wn_ref[...], an_ref[...], hn,
               wan_ref, wmn_ref, an1_ref, cn_ref, bn1_ref, dn_ref)
    _proj_side(xe_ref[...], we_ref[...], ae_ref[...], he,
               wae_ref, wme_ref, ae1_ref, ce_ref, be_ref, de_ref)


def _side_shapes(r, h):
    return (
        jax.ShapeDtypeStruct((r, h + 1), jnp.float32),
        jax.ShapeDtypeStruct((1, h), jnp.float32),
        jax.ShapeDtypeStruct((r, 1), jnp.float32),
        jax.ShapeDtypeStruct((r, 1), jnp.float32),
        jax.ShapeDtypeStruct((r, 1), jnp.float32),
        jax.ShapeDtypeStruct((r, 1), jnp.float32),
    )


def _proj(xn, xe, p):
    rn = xn.shape[0]
    re = xe.shape[0]
    hn = p['Wn'].shape[1]
    he = p['We'].shape[1]
    return pl.pallas_call(
        functools.partial(_proj_body, hn, he),
        out_shape=_side_shapes(rn, hn) + _side_shapes(re, he),
    )(xn, p['Wn'], p['an'], xe, p['We'], p['ae'])


def _bn_relu_expr(z, s, ss, gamma, beta, n):
    mu = s / n
    var = ss / n - mu * mu
    return jnp.maximum((z - mu) / jnp.sqrt(var + _EPS) * gamma + beta, 0.0)


def _projbn_body(hn, he, nn, ne,
                 zn_ref, sn_ref, ssn_ref, ze_ref, se_ref, sse_ref,
                 g_ref, bt_ref, wn_ref, an_ref, we_ref, ae_ref,
                 wan_ref, wmn_ref, an1_ref, cn_ref, bn1_ref, dn_ref,
                 wae_ref, wme_ref, ae1_ref, ce_ref, be_ref, de_ref):
    xn = _bn_relu_expr(zn_ref[...], sn_ref[...], ssn_ref[...],
                       g_ref[...], bt_ref[...], nn)
    xe = _bn_relu_expr(ze_ref[...], se_ref[...], sse_ref[...],
                       g_ref[...], bt_ref[...], ne)
    _proj_side(xn, wn_ref[...], an_ref[...], hn,
               wan_ref, wmn_ref, an1_ref, cn_ref, bn1_ref, dn_ref)
    _proj_side(xe, we_ref[...], ae_ref[...], he,
               wae_ref, wme_ref, ae1_ref, ce_ref, be_ref, de_ref)


def _projbn(zn, sn, ssn, ze, se, sse, bp, p):
    rn = zn.shape[0]
    re = ze.shape[0]
    hn = p['Wn'].shape[1]
    he = p['We'].shape[1]
    return pl.pallas_call(
        functools.partial(_projbn_body, hn, he, float(rn), float(re)),
        out_shape=_side_shapes(rn, hn) + _side_shapes(re, he),
    )(zn, sn, ssn, ze, se, sse,
      bp['gamma'].reshape(1, -1), bp['beta'].reshape(1, -1),
      p['Wn'], p['an'], p['We'], p['ae'])


def _att_num(mask_ref, a_ref, c_ref, b_ref, d_ref):
    maskf = mask_ref[...].astype(jnp.float32)
    return maskf * jnp.maximum(a_ref[...] * b_ref[...],
                               c_ref[...] * d_ref[...])


def _att_out(p, w_ref, wm_ref, h):
    ha = jnp.dot(p, w_ref[...], preferred_element_type=jnp.float32)
    den = ha[:, h:h + 1]
    return jnp.where(den > 0, ha[:, :h] / den, wm_ref[...])


def _att_body(h, final, mask_ref, a_ref, c_ref, b_ref, d_ref, w_ref,
              wm_ref, o_ref):
    out = _att_out(_att_num(mask_ref, a_ref, c_ref, b_ref, d_ref),
                   w_ref, wm_ref, h)
    if final:
        out = _elu(out)
        out = out - jnp.max(out, axis=1, keepdims=True)
        out = out - jnp.log(jnp.sum(jnp.exp(out), axis=1, keepdims=True))
    o_ref[...] = out


def _att(mask8, vecs, wa, wm, final=False):
    a, c, b, d = vecs
    r, cdim = mask8.shape
    h = wa.shape[1] - 1
    return pl.pallas_call(
        functools.partial(_att_body, h, final),
        grid=(r // _BLK,),
        in_specs=[
            pl.BlockSpec((_BLK, cdim), lambda i: (i, 0)),
            pl.BlockSpec((_BLK, 1), lambda i: (i, 0)),
            pl.BlockSpec((_BLK, 1), lambda i: (i, 0)),
            pl.BlockSpec((1, cdim), lambda i: (0, 0)),
            pl.BlockSpec((1, cdim), lambda i: (0, 0)),
            pl.BlockSpec((cdim, h + 1), lambda i: (0, 0)),
            pl.BlockSpec((1, h), lambda i: (0, 0)),
        ],
        out_specs=pl.BlockSpec((_BLK, h), lambda i: (i, 0)),
        out_shape=jax.ShapeDtypeStruct((r, h), jnp.float32),
    )(mask8, a, c, b.reshape(1, -1), d.reshape(1, -1), wa, wm)


def _att_cat_body(h, mask_ref, a_ref, c_ref, b_ref, d_ref, w_ref, wm_ref,
                  nec_ref, hp_ref, ep_ref, eo_ref):
    ep = _att_out(_att_num(mask_ref, a_ref, c_ref, b_ref, d_ref),
                  w_ref, wm_ref, h)
    ep_ref[...] = ep
    pooled = jax.lax.dot_general(
        nec_ref[...].astype(jnp.float32), hp_ref[...],
        (((0,), (0,)), ((), ())), preferred_element_type=jnp.float32)
    eo_ref[:, :h] = _elu(ep)
    eo_ref[:, h:] = _elu(pooled)


def _att_cat(mask8, vecs, wa, wm, nea8, hp):
    a, c, b, d = vecs
    r, cdim = mask8.shape
    n = nea8.shape[0]
    h = wa.shape[1] - 1
    return pl.pallas_call(
        functools.partial(_att_cat_body, h),
        grid=(r // _BLK,),
        in_specs=[
            pl.BlockSpec((_BLK, cdim), lambda i: (i, 0)),
            pl.BlockSpec((_BLK, 1), lambda i: (i, 0)),
            pl.BlockSpec((_BLK, 1), lambda i: (i, 0)),
            pl.BlockSpec((1, cdim), lambda i: (0, 0)),
            pl.BlockSpec((1, cdim), lambda i: (0, 0)),
            pl.BlockSpec((cdim, h + 1), lambda i: (0, 0)),
            pl.BlockSpec((1, h), lambda i: (0, 0)),
            pl.BlockSpec((n, _BLK), lambda i: (0, i)),
            pl.BlockSpec((n, h), lambda i: (0, 0)),
        ],
        out_specs=(
            pl.BlockSpec((_BLK, h), lambda i: (i, 0)),
            pl.BlockSpec((_BLK, 2 * h), lambda i: (i, 0)),
        ),
        out_shape=(
            jax.ShapeDtypeStruct((r, h), jnp.float32),
            jax.ShapeDtypeStruct((r, 2 * h), jnp.float32),
        ),
    )(mask8, a, c, b.reshape(1, -1), d.reshape(1, -1), wa, wm, nea8, hp)


def _cat_body(h, ne_ref, hp_ref, ep_ref, o_ref):
    pooled = jnp.dot(ne_ref[...].astype(jnp.float32), ep_ref[...],
                     preferred_element_type=jnp.float32)
    o_ref[:, :h] = _elu(hp_ref[...])
    o_ref[:, h:] = _elu(pooled)


def _cat(ne8, hp, ep):
    r, cdim = ne8.shape
    h = hp.shape[1]
    return pl.pallas_call(
        functools.partial(_cat_body, h),
        grid=(r // _BLK,),
        in_specs=[
            pl.BlockSpec((_BLK, cdim), lambda i: (i, 0)),
            pl.BlockSpec((_BLK, h), lambda i: (i, 0)),
            pl.BlockSpec((cdim, h), lambda i: (0, 0)),
        ],
        out_specs=pl.BlockSpec((_BLK, 2 * h), lambda i: (i, 0)),
        out_shape=jax.ShapeDtypeStruct((r, 2 * h), jnp.float32),
    )(ne8, hp, ep)


def _mlp_stats(pooled, w1_ref, b1_ref, w2_ref, b2_ref, z_ref, s_ref, ss_ref):
    t = jnp.maximum(
        jnp.dot(pooled, w1_ref[...], preferred_element_type=jnp.float32)
        + b1_ref[...], 0.0)
    z = jnp.dot(t, w2_ref[...], preferred_element_type=jnp.float32) + b2_ref[...]
    z_ref[...] = z

    @pl.when(pl.program_id(0) == 0)
    def _init():
        s_ref[...] = jnp.zeros_like(s_ref)
        ss_ref[...] = jnp.zeros_like(ss_ref)

    s_ref[...] += jnp.sum(z, axis=0, keepdims=True)
    ss_ref[...] += jnp.sum(z * z, axis=0, keepdims=True)


def _pool1_body(a_ref, h_ref, w1_ref, b1_ref, w2_ref, b2_ref,
                z_ref, s_ref, ss_ref):
    pooled = jnp.dot(a_ref[...].astype(jnp.float32), h_ref[...],
                     preferred_element_type=jnp.float32)
    _mlp_stats(pooled, w1_ref, b1_ref, w2_ref, b2_ref, z_ref, s_ref, ss_ref)


def _pool2_body(n, a_ref, zp_ref, sp_ref, ssp_ref, g_ref, bt_ref,
                w1_ref, b1_ref, w2_ref, b2_ref, z_ref, s_ref, ss_ref):
    h = _bn_relu_expr(zp_ref[...], sp_ref[...], ssp_ref[...],
                      g_ref[...], bt_ref[...], n)
    pooled = jnp.dot(a_ref[...].astype(jnp.float32), h,
                     preferred_element_type=jnp.float32)
    _mlp_stats(pooled, w1_ref, b1_ref, w2_ref, b2_ref, z_ref, s_ref, ss_ref)


def _pool_outs(r, k2):
    return (
        (
            pl.BlockSpec((_BLK, k2), lambda i: (i, 0)),
            pl.BlockSpec((1, k2), lambda i: (0, 0)),
            pl.BlockSpec((1, k2), lambda i: (0, 0)),
        ),
        (
            jax.ShapeDtypeStruct((r, k2), jnp.float32),
            jax.ShapeDtypeStruct((1, k2), jnp.float32),
            jax.ShapeDtypeStruct((1, k2), jnp.float32),
        ),
    )


def _pool1(a8, h, mp):
    r, cdim = a8.shape
    f = h.shape[1]
    k1 = mp['W1'].shape[1]
    k2 = mp['W2'].shape[1]
    out_specs, out_shape = _pool_outs(r, k2)
    return pl.pallas_call(
        _pool1_body,
        grid=(r // _BLK,),
        in_specs=[
            pl.BlockSpec((_BLK, cdim), lambda i: (i, 0)),
            pl.BlockSpec((cdim, f), lambda i: (0, 0)),
            pl.BlockSpec((f, k1), lambda i: (0, 0)),
            pl.BlockSpec((1, k1), lambda i: (0, 0)),
            pl.BlockSpec((k1, k2), lambda i: (0, 0)),
            pl.BlockSpec((1, k2), lambda i: (0, 0)),
        ],
        out_specs=out_specs,
        out_shape=out_shape,
    )(a8, h, mp['W1'], mp['b1'].reshape(1, -1), mp['W2'],
      mp['b2'].reshape(1, -1))


def _pool2(a8, zp, sp, ssp, bp, mp):
    r, cdim = a8.shape
    f = zp.shape[1]
    k1 = mp['W1'].shape[1]
    k2 = mp['W2'].shape[1]
    out_specs, out_shape = _pool_outs(r, k2)
    return pl.pallas_call(
        functools.partial(_pool2_body, float(cdim)),
        grid=(r // _BLK,),
        in_specs=[
            pl.BlockSpec((_BLK, cdim), lambda i: (i, 0)),
            pl.BlockSpec((cdim, f), lambda i: (0, 0)),
            pl.BlockSpec((1, f), lambda i: (0, 0)),
            pl.BlockSpec((1, f), lambda i: (0, 0)),
            pl.BlockSpec((1, f), lambda i: (0, 0)),
            pl.BlockSpec((1, f), lambda i: (0, 0)),
            pl.BlockSpec((f, k1), lambda i: (0, 0)),
            pl.BlockSpec((1, k1), lambda i: (0, 0)),
            pl.BlockSpec((k1, k2), lambda i: (0, 0)),
            pl.BlockSpec((1, k2), lambda i: (0, 0)),
        ],
        out_specs=out_specs,
        out_shape=out_shape,
    )(a8, zp, sp, ssp, bp['gamma'].reshape(1, -1), bp['beta'].reshape(1, -1),
      mp['W1'], mp['b1'].reshape(1, -1), mp['W2'], mp['b2'].reshape(1, -1))


def kernel(x, e_x, adj, e_adj, n_e_adj, params):
    adj8 = adj.astype(jnp.int8)
    eadj8 = e_adj.astype(jnp.int8)
    nea8 = n_e_adj.astype(jnp.int8)

    xn, xe = x, e_x
    for lp in (params['in_att'], params['att0']):
        (wan, wmn, av, cv, bv, dv,
         wae, wme, ae_, ce, be, de) = _proj(xn, xe, lp)
        hp = _att(adj8, (av, cv, bv, dv), wan, wmn)
        ep, eo = _att_cat(eadj8, (ae_, ce, be, de), wae, wme, nea8, hp)
        xo = _cat(nea8, hp, ep)
        xn, xe = xo, eo

    zn, sn, ssn = _pool1(adj8, xn, params['mlp0'])
    ze, se, sse = _pool1(eadj8, xe, params['mlp0'])
    zn, sn2, ssn2 = _pool2(adj8, zn, sn, ssn, params['bn0'], params['mlp1'])
    ze, se2, sse2 = _pool2(eadj8, ze, se, sse, params['bn0'], params['mlp1'])

    (wan, wmn, av, cv, bv, dv,
     wae, wme, ae_, ce, be, de) = _projbn(zn, sn2, ssn2, ze, se2, sse2,
                                          params['bn1'], params['out_att'])
    fx = _att(adj8, (av, cv, bv, dv), wan, wmn, final=True)
    fe = _att(eadj8, (ae_, ce, be, de), wae, wme, final=True)
    return fx, fe


# speedup vs baseline: 1.7384x; 1.2043x over previous
import functools

import jax
import jax.numpy as jnp
from jax.experimental import pallas as pl

_ALPHA = 0.2
_BLK = 256
_EPS = 1e-5


def _abcd(wh, an, h):
    f1 = jax.lax.dot_general(an[:h, :], wh, (((0,), (1,)), ((), ())),
                             preferred_element_type=jnp.float32)
    f2 = jax.lax.dot_general(an[h:, :], wh, (((0,), (1,)), ((), ())),
                             preferred_element_type=jnp.float32)
    mf2 = jnp.max(f2)
    t = f1 + mf2
    m = jnp.maximum(t, _ALPHA * t)
    a = jnp.exp(t - m)
    c = jnp.exp(_ALPHA * t - m)
    b = jnp.exp(f2 - mf2)
    d = jnp.exp(_ALPHA * (f2 - mf2))
    return a, c, b, d


def _elu(x):
    return jnp.where(x > 0, x, jnp.exp(x) - 1.0)


def _proj_side(x, wn, an, h, wa_ref, wm_ref, a_ref, c_ref, b_ref, d_ref):
    wh = jnp.dot(x, wn, preferred_element_type=jnp.float32)
    wa_ref[:, :h] = wh
    wa_ref[:, h:] = jnp.ones_like(wa_ref[:, h:])
    wm_ref[...] = jnp.mean(wh, axis=0, keepdims=True)
    a, c, b, d = _abcd(wh, an, h)
    a_ref[...] = a
    c_ref[...] = c
    b_ref[...] = b
    d_ref[...] = d


def _proj_body(hn, he, xn_ref, wn_ref, an_ref, xe_ref, we_ref, ae_ref,
               wan_ref, wmn_ref, an1_ref, cn_ref, bn1_ref, dn_ref,
               wae_ref, wme_ref, ae1_ref, ce_ref, be_ref, de_ref):
    _proj_side(xn_ref[...], wn_ref[...], an_ref[...], hn,
               wan_ref, wmn_ref, an1_ref, cn_ref, bn1_ref, dn_ref)
    _proj_side(xe_ref[...], we_ref[...], ae_ref[...], he,
               wae_ref, wme_ref, ae1_ref, ce_ref, be_ref, de_ref)


def _side_shapes(r, h):
    return (
        jax.ShapeDtypeStruct((r, h + 1), jnp.float32),
        jax.ShapeDtypeStruct((1, h), jnp.float32),
        jax.ShapeDtypeStruct((1, r), jnp.float32),
        jax.ShapeDtypeStruct((1, r), jnp.float32),
        jax.ShapeDtypeStruct((1, r), jnp.float32),
        jax.ShapeDtypeStruct((1, r), jnp.float32),
    )


def _proj(xn, xe, p):
    rn = xn.shape[0]
    re = xe.shape[0]
    hn = p['Wn'].shape[1]
    he = p['We'].shape[1]
    return pl.pallas_call(
        functools.partial(_proj_body, hn, he),
        out_shape=_side_shapes(rn, hn) + _side_shapes(re, he),
    )(xn, p['Wn'], p['an'], xe, p['We'], p['ae'])


def _bn_relu_expr(z, s, ss, gamma, beta, n):
    mu = s / n
    var = ss / n - mu * mu
    return jnp.maximum((z - mu) / jnp.sqrt(var + _EPS) * gamma + beta, 0.0)


def _projbn_body(hn, he, nn, ne,
                 zn_ref, sn_ref, ssn_ref, ze_ref, se_ref, sse_ref,
                 g_ref, bt_ref, wn_ref, an_ref, we_ref, ae_ref,
                 wan_ref, wmn_ref, an1_ref, cn_ref, bn1_ref, dn_ref,
                 wae_ref, wme_ref, ae1_ref, ce_ref, be_ref, de_ref):
    xn = _bn_relu_expr(zn_ref[...], sn_ref[...], ssn_ref[...],
                       g_ref[...], bt_ref[...], nn)
    xe = _bn_relu_expr(ze_ref[...], se_ref[...], sse_ref[...],
                       g_ref[...], bt_ref[...], ne)
    _proj_side(xn, wn_ref[...], an_ref[...], hn,
               wan_ref, wmn_ref, an1_ref, cn_ref, bn1_ref, dn_ref)
    _proj_side(xe, we_ref[...], ae_ref[...], he,
               wae_ref, wme_ref, ae1_ref, ce_ref, be_ref, de_ref)


def _projbn(zn, sn, ssn, ze, se, sse, bp, p):
    rn = zn.shape[0]
    re = ze.shape[0]
    hn = p['Wn'].shape[1]
    he = p['We'].shape[1]
    return pl.pallas_call(
        functools.partial(_projbn_body, hn, he, float(rn), float(re)),
        out_shape=_side_shapes(rn, hn) + _side_shapes(re, he),
    )(zn, sn, ssn, ze, se, sse,
      bp['gamma'].reshape(1, -1), bp['beta'].reshape(1, -1),
      p['Wn'], p['an'], p['We'], p['ae'])


def _att_num(mask_ref, a_ref, c_ref, b_ref, d_ref):
    maskf = mask_ref[...].astype(jnp.float32)
    at = jnp.transpose(a_ref[...])
    ct = jnp.transpose(c_ref[...])
    return maskf * jnp.maximum(at * b_ref[...], ct * d_ref[...])


def _att_out(p, w_ref, wm_ref, h):
    ha = jnp.dot(p, w_ref[...], preferred_element_type=jnp.float32)
    den = ha[:, h:h + 1]
    return jnp.where(den > 0, ha[:, :h] / den, wm_ref[...])


def _att_body(h, final, cast, mask_ref, a_ref, c_ref, b_ref, d_ref, w_ref,
              wm_ref, o_ref, *rest):
    out = _att_out(_att_num(mask_ref, a_ref, c_ref, b_ref, d_ref),
                   w_ref, wm_ref, h)
    if final:
        out = _elu(out)
        out = out - jnp.max(out, axis=1, keepdims=True)
        out = out - jnp.log(jnp.sum(jnp.exp(out), axis=1, keepdims=True))
    o_ref[...] = out
    if cast:
        rest[0][...] = mask_ref[...].astype(jnp.int8)


def _att(mask, vecs, wa, wm, final=False, cast=False):
    a, c, b, d = vecs
    r, cdim = mask.shape
    h = wa.shape[1] - 1
    out_specs = [pl.BlockSpec((_BLK, h), lambda i: (i, 0))]
    out_shape = [jax.ShapeDtypeStruct((r, h), jnp.float32)]
    if cast:
        out_specs.append(pl.BlockSpec((_BLK, cdim), lambda i: (i, 0)))
        out_shape.append(jax.ShapeDtypeStruct((r, cdim), jnp.int8))
    return pl.pallas_call(
        functools.partial(_att_body, h, final, cast),
        grid=(r // _BLK,),
        in_specs=[
            pl.BlockSpec((_BLK, cdim), lambda i: (i, 0)),
            pl.BlockSpec((1, _BLK), lambda i: (0, i)),
            pl.BlockSpec((1, _BLK), lambda i: (0, i)),
            pl.BlockSpec((1, cdim), lambda i: (0, 0)),
            pl.BlockSpec((1, cdim), lambda i: (0, 0)),
            pl.BlockSpec((cdim, h + 1), lambda i: (0, 0)),
            pl.BlockSpec((1, h), lambda i: (0, 0)),
        ],
        out_specs=tuple(out_specs),
        out_shape=tuple(out_shape),
    )(mask, a, c, b, d, wa, wm)


def _att_cat_body(h, cast, mask_ref, a_ref, c_ref, b_ref, d_ref, w_ref,
                  wm_ref, nec_ref, hp_ref, ep_ref, eo_ref, *rest):
    ep = _att_out(_att_num(mask_ref, a_ref, c_ref, b_ref, d_ref),
                  w_ref, wm_ref, h)
    ep_ref[...] = ep
    pooled = jax.lax.dot_general(
        nec_ref[...].astype(jnp.float32), hp_ref[...],
        (((0,), (0,)), ((), ())), preferred_element_type=jnp.float32)
    eo_ref[:, :h] = _elu(ep)
    eo_ref[:, h:] = _elu(pooled)
    if cast:
        rest[0][...] = mask_ref[...].astype(jnp.int8)


def _att_cat(mask, vecs, wa, wm, nea8, hp, cast=False):
    a, c, b, d = vecs
    r, cdim = mask.shape
    n = nea8.shape[0]
    h = wa.shape[1] - 1
    out_specs = [
        pl.BlockSpec((_BLK, h), lambda i: (i, 0)),
        pl.BlockSpec((_BLK, 2 * h), lambda i: (i, 0)),
    ]
    out_shape = [
        jax.ShapeDtypeStruct((r, h), jnp.float32),
        jax.ShapeDtypeStruct((r, 2 * h), jnp.float32),
    ]
    if cast:
        out_specs.append(pl.BlockSpec((_BLK, cdim), lambda i: (i, 0)))
        out_shape.append(jax.ShapeDtypeStruct((r, cdim), jnp.int8))
    return pl.pallas_call(
        functools.partial(_att_cat_body, h, cast),
        grid=(r // _BLK,),
        in_specs=[
            pl.BlockSpec((_BLK, cdim), lambda i: (i, 0)),
            pl.BlockSpec((1, _BLK), lambda i: (0, i)),
            pl.BlockSpec((1, _BLK), lambda i: (0, i)),
            pl.BlockSpec((1, cdim), lambda i: (0, 0)),
            pl.BlockSpec((1, cdim), lambda i: (0, 0)),
            pl.BlockSpec((cdim, h + 1), lambda i: (0, 0)),
            pl.BlockSpec((1, h), lambda i: (0, 0)),
            pl.BlockSpec((n, _BLK), lambda i: (0, i)),
            pl.BlockSpec((n, h), lambda i: (0, 0)),
        ],
        out_specs=tuple(out_specs),
        out_shape=tuple(out_shape),
    )(mask, a, c, b, d, wa, wm, nea8, hp)


def _cat_body(h, ne_ref, hp_ref, ep_ref, o_ref):
    pooled = jnp.dot(ne_ref[...].astype(jnp.float32), ep_ref[...],
                     preferred_element_type=jnp.float32)
    o_ref[:, :h] = _elu(hp_ref[...])
    o_ref[:, h:] = _elu(pooled)


def _cat(ne8, hp, ep):
    r, cdim = ne8.shape
    h = hp.shape[1]
    return pl.pallas_call(
        functools.partial(_cat_body, h),
        grid=(r // _BLK,),
        in_specs=[
            pl.BlockSpec((_BLK, cdim), lambda i: (i, 0)),
            pl.BlockSpec((_BLK, h), lambda i: (i, 0)),
            pl.BlockSpec((cdim, h), lambda i: (0, 0)),
        ],
        out_specs=pl.BlockSpec((_BLK, 2 * h), lambda i: (i, 0)),
        out_shape=jax.ShapeDtypeStruct((r, 2 * h), jnp.float32),
    )(ne8, hp, ep)


def _mlp_stats(pooled, w1_ref, b1_ref, w2_ref, b2_ref, z_ref, s_ref, ss_ref):
    t = jnp.maximum(
        jnp.dot(pooled, w1_ref[...], preferred_element_type=jnp.float32)
        + b1_ref[...], 0.0)
    z = jnp.dot(t, w2_ref[...], preferred_element_type=jnp.float32) + b2_ref[...]
    z_ref[...] = z

    @pl.when(pl.program_id(0) == 0)
    def _init():
        s_ref[...] = jnp.zeros_like(s_ref)
        ss_ref[...] = jnp.zeros_like(ss_ref)

    s_ref[...] += jnp.sum(z, axis=0, keepdims=True)
    ss_ref[...] += jnp.sum(z * z, axis=0, keepdims=True)


def _pool1_body(a_ref, h_ref, w1_ref, b1_ref, w2_ref, b2_ref,
                z_ref, s_ref, ss_ref):
    pooled = jnp.dot(a_ref[...].astype(jnp.float32), h_ref[...],
                     preferred_element_type=jnp.float32)
    _mlp_stats(pooled, w1_ref, b1_ref, w2_ref, b2_ref, z_ref, s_ref, ss_ref)


def _pool2_body(n, a_ref, zp_ref, sp_ref, ssp_ref, g_ref, bt_ref,
                w1_ref, b1_ref, w2_ref, b2_ref, z_ref, s_ref, ss_ref):
    h = _bn_relu_expr(zp_ref[...], sp_ref[...], ssp_ref[...],
                      g_ref[...], bt_ref[...], n)
    pooled = jnp.dot(a_ref[...].astype(jnp.float32), h,
                     preferred_element_type=jnp.float32)
    _mlp_stats(pooled, w1_ref, b1_ref, w2_ref, b2_ref, z_ref, s_ref, ss_ref)


def _pool_outs(r, k2):
    return (
        (
            pl.BlockSpec((_BLK, k2), lambda i: (i, 0)),
            pl.BlockSpec((1, k2), lambda i: (0, 0)),
            pl.BlockSpec((1, k2), lambda i: (0, 0)),
        ),
        (
            jax.ShapeDtypeStruct((r, k2), jnp.float32),
            jax.ShapeDtypeStruct((1, k2), jnp.float32),
            jax.ShapeDtypeStruct((1, k2), jnp.float32),
        ),
    )


def _pool1(a8, h, mp):
    r, cdim = a8.shape
    f = h.shape[1]
    k1 = mp['W1'].shape[1]
    k2 = mp['W2'].shape[1]
    out_specs, out_shape = _pool_outs(r, k2)
    return pl.pallas_call(
        _pool1_body,
        grid=(r // _BLK,),
        in_specs=[
            pl.BlockSpec((_BLK, cdim), lambda i: (i, 0)),
            pl.BlockSpec((cdim, f), lambda i: (0, 0)),
            pl.BlockSpec((f, k1), lambda i: (0, 0)),
            pl.BlockSpec((1, k1), lambda i: (0, 0)),
            pl.BlockSpec((k1, k2), lambda i: (0, 0)),
            pl.BlockSpec((1, k2), lambda i: (0, 0)),
        ],
        out_specs=out_specs,
        out_shape=out_shape,
    )(a8, h, mp['W1'], mp['b1'].reshape(1, -1), mp['W2'],
      mp['b2'].reshape(1, -1))


def _pool2(a8, zp, sp, ssp, bp, mp):
    r, cdim = a8.shape
    f = zp.shape[1]
    k1 = mp['W1'].shape[1]
    k2 = mp['W2'].shape[1]
    out_specs, out_shape = _pool_outs(r, k2)
    return pl.pallas_call(
        functools.partial(_pool2_body, float(cdim)),
        grid=(r // _BLK,),
        in_specs=[
            pl.BlockSpec((_BLK, cdim), lambda i: (i, 0)),
            pl.BlockSpec((cdim, f), lambda i: (0, 0)),
            pl.BlockSpec((1, f), lambda i: (0, 0)),
            pl.BlockSpec((1, f), lambda i: (0, 0)),
            pl.BlockSpec((1, f), lambda i: (0, 0)),
            pl.BlockSpec((1, f), lambda i: (0, 0)),
            pl.BlockSpec((f, k1), lambda i: (0, 0)),
            pl.BlockSpec((1, k1), lambda i: (0, 0)),
            pl.BlockSpec((k1, k2), lambda i: (0, 0)),
            pl.BlockSpec((1, k2), lambda i: (0, 0)),
        ],
        out_specs=out_specs,
        out_shape=out_shape,
    )(a8, zp, sp, ssp, bp['gamma'].reshape(1, -1), bp['beta'].reshape(1, -1),
      mp['W1'], mp['b1'].reshape(1, -1), mp['W2'], mp['b2'].reshape(1, -1))


def kernel(x, e_x, adj, e_adj, n_e_adj, params):
    nea8 = n_e_adj.astype(jnp.int8)

    xn, xe = x, e_x
    for li, lp in enumerate((params['in_att'], params['att0'])):
        (wan, wmn, av, cv, bv, dv,
         wae, wme, ae_, ce, be, de) = _proj(xn, xe, lp)
        if li == 0:
            hp, adj8 = _att(adj, (av, cv, bv, dv), wan, wmn, cast=True)
            ep, eo, eadj8 = _att_cat(e_adj, (ae_, ce, be, de), wae, wme,
                                     nea8, hp, cast=True)
        else:
            hp, = _att(adj8, (av, cv, bv, dv), wan, wmn)
            ep, eo = _att_cat(eadj8, (ae_, ce, be, de), wae, wme, nea8, hp)
        xo = _cat(nea8, hp, ep)
        xn, xe = xo, eo

    zn, sn, ssn = _pool1(adj8, xn, params['mlp0'])
    ze, se, sse = _pool1(eadj8, xe, params['mlp0'])
    zn, sn2, ssn2 = _pool2(adj8, zn, sn, ssn, params['bn0'], params['mlp1'])
    ze, se2, sse2 = _pool2(eadj8, ze, se, sse, params['bn0'], params['mlp1'])

    (wan, wmn, av, cv, bv, dv,
     wae, wme, ae_, ce, be, de) = _projbn(zn, sn2, ssn2, ze, se2, sse2,
                                          params['bn1'], params['out_att'])
    fx, = _att(adj8, (av, cv, bv, dv), wan, wmn, final=True)
    fe, = _att(eadj8, (ae_, ce, be, de), wae, wme, final=True)
    return fx, fe


# speedup vs baseline: 1.9908x; 1.1452x over previous
import functools

import jax
import jax.numpy as jnp
from jax.experimental import pallas as pl

_ALPHA = 0.2
_BLK = 512
_EPS = 1e-5


def _abcd(wh, an, h):
    f1 = jax.lax.dot_general(an[:h, :], wh, (((0,), (1,)), ((), ())),
                             preferred_element_type=jnp.float32)
    f2 = jax.lax.dot_general(an[h:, :], wh, (((0,), (1,)), ((), ())),
                             preferred_element_type=jnp.float32)
    mf2 = jnp.max(f2)
    t = f1 + mf2
    m = jnp.maximum(t, _ALPHA * t)
    a = jnp.exp(t - m)
    c = jnp.exp(_ALPHA * t - m)
    b = jnp.exp(f2 - mf2)
    d = jnp.exp(_ALPHA * (f2 - mf2))
    return a, c, b, d


def _elu(x):
    return jnp.where(x > 0, x, jnp.exp(x) - 1.0)


def _proj_side(x, wn, an, h, wa_ref, wm_ref, a_ref, c_ref, b_ref, d_ref):
    wh = jnp.dot(x, wn, preferred_element_type=jnp.float32)
    wa_ref[:, :h] = wh
    wa_ref[:, h:] = jnp.ones_like(wa_ref[:, h:])
    wm_ref[...] = jnp.mean(wh, axis=0, keepdims=True)
    a, c, b, d = _abcd(wh, an, h)
    a_ref[...] = a
    c_ref[...] = c
    b_ref[...] = b
    d_ref[...] = d


def _proj_body(hn, he, xn_ref, wn_ref, an_ref, xe_ref, we_ref, ae_ref,
               wan_ref, wmn_ref, an1_ref, cn_ref, bn1_ref, dn_ref,
               wae_ref, wme_ref, ae1_ref, ce_ref, be_ref, de_ref):
    _proj_side(xn_ref[...], wn_ref[...], an_ref[...], hn,
               wan_ref, wmn_ref, an1_ref, cn_ref, bn1_ref, dn_ref)
    _proj_side(xe_ref[...], we_ref[...], ae_ref[...], he,
               wae_ref, wme_ref, ae1_ref, ce_ref, be_ref, de_ref)


def _side_shapes(r, h):
    return (
        jax.ShapeDtypeStruct((r, h + 1), jnp.float32),
        jax.ShapeDtypeStruct((1, h), jnp.float32),
        jax.ShapeDtypeStruct((1, r), jnp.float32),
        jax.ShapeDtypeStruct((1, r), jnp.float32),
        jax.ShapeDtypeStruct((1, r), jnp.float32),
        jax.ShapeDtypeStruct((1, r), jnp.float32),
    )


def _proj(xn, xe, p):
    rn = xn.shape[0]
    re = xe.shape[0]
    hn = p['Wn'].shape[1]
    he = p['We'].shape[1]
    return pl.pallas_call(
        functools.partial(_proj_body, hn, he),
        out_shape=_side_shapes(rn, hn) + _side_shapes(re, he),
    )(xn, p['Wn'], p['an'], xe, p['We'], p['ae'])


def _bn_relu_expr(z, s, ss, gamma, beta, n):
    mu = s / n
    var = ss / n - mu * mu
    return jnp.maximum((z - mu) / jnp.sqrt(var + _EPS) * gamma + beta, 0.0)


def _projbn_body(hn, he, nn, ne,
                 zn_ref, sn_ref, ssn_ref, ze_ref, se_ref, sse_ref,
                 g_ref, bt_ref, wn_ref, an_ref, we_ref, ae_ref,
                 wan_ref, wmn_ref, an1_ref, cn_ref, bn1_ref, dn_ref,
                 wae_ref, wme_ref, ae1_ref, ce_ref, be_ref, de_ref):
    xn = _bn_relu_expr(zn_ref[...], sn_ref[...], ssn_ref[...],
                       g_ref[...], bt_ref[...], nn)
    xe = _bn_relu_expr(ze_ref[...], se_ref[...], sse_ref[...],
                       g_ref[...], bt_ref[...], ne)
    _proj_side(xn, wn_ref[...], an_ref[...], hn,
               wan_ref, wmn_ref, an1_ref, cn_ref, bn1_ref, dn_ref)
    _proj_side(xe, we_ref[...], ae_ref[...], he,
               wae_ref, wme_ref, ae1_ref, ce_ref, be_ref, de_ref)


def _projbn(zn, sn, ssn, ze, se, sse, bp, p):
    rn = zn.shape[0]
    re = ze.shape[0]
    hn = p['Wn'].shape[1]
    he = p['We'].shape[1]
    return pl.pallas_call(
        functools.partial(_projbn_body, hn, he, float(rn), float(re)),
        out_shape=_side_shapes(rn, hn) + _side_shapes(re, he),
    )(zn, sn, ssn, ze, se, sse,
      bp['gamma'].reshape(1, -1), bp['beta'].reshape(1, -1),
      p['Wn'], p['an'], p['We'], p['ae'])


def _att_num(mask_ref, a_ref, c_ref, b_ref, d_ref):
    maskf = mask_ref[...].astype(jnp.float32)
    at = jnp.transpose(a_ref[...])
    ct = jnp.transpose(c_ref[...])
    return maskf * jnp.maximum(at * b_ref[...], ct * d_ref[...])


def _att_out(p, w_ref, wm_ref, h):
    ha = jnp.dot(p, w_ref[...], preferred_element_type=jnp.float32)
    den = ha[:, h:h + 1]
    return jnp.where(den > 0, ha[:, :h] / den, wm_ref[...])


def _att_body(h, final, cast, mask_ref, a_ref, c_ref, b_ref, d_ref, w_ref,
              wm_ref, o_ref, *rest):
    out = _att_out(_att_num(mask_ref, a_ref, c_ref, b_ref, d_ref),
                   w_ref, wm_ref, h)
    if final:
        out = _elu(out)
        out = out - jnp.max(out, axis=1, keepdims=True)
        out = out - jnp.log(jnp.sum(jnp.exp(out), axis=1, keepdims=True))
    o_ref[...] = out
    if cast:
        rest[0][...] = mask_ref[...].astype(jnp.int8)


def _att(mask, vecs, wa, wm, final=False, cast=False):
    a, c, b, d = vecs
    r, cdim = mask.shape
    h = wa.shape[1] - 1
    out_specs = [pl.BlockSpec((_BLK, h), lambda i: (i, 0))]
    out_shape = [jax.ShapeDtypeStruct((r, h), jnp.float32)]
    if cast:
        out_specs.append(pl.BlockSpec((_BLK, cdim), lambda i: (i, 0)))
        out_shape.append(jax.ShapeDtypeStruct((r, cdim), jnp.int8))
    return pl.pallas_call(
        functools.partial(_att_body, h, final, cast),
        grid=(r // _BLK,),
        in_specs=[
            pl.BlockSpec((_BLK, cdim), lambda i: (i, 0)),
            pl.BlockSpec((1, _BLK), lambda i: (0, i)),
            pl.BlockSpec((1, _BLK), lambda i: (0, i)),
            pl.BlockSpec((1, cdim), lambda i: (0, 0)),
            pl.BlockSpec((1, cdim), lambda i: (0, 0)),
            pl.BlockSpec((cdim, h + 1), lambda i: (0, 0)),
            pl.BlockSpec((1, h), lambda i: (0, 0)),
        ],
        out_specs=tuple(out_specs),
        out_shape=tuple(out_shape),
    )(mask, a, c, b, d, wa, wm)


def _att_cat_body(h, cast, mask_ref, a_ref, c_ref, b_ref, d_ref, w_ref,
                  wm_ref, nec_ref, hp_ref, ep_ref, eo_ref, *rest):
    ep = _att_out(_att_num(mask_ref, a_ref, c_ref, b_ref, d_ref),
                  w_ref, wm_ref, h)
    ep_ref[...] = ep
    pooled = jax.lax.dot_general(
        nec_ref[...].astype(jnp.float32), hp_ref[...],
        (((0,), (0,)), ((), ())), preferred_element_type=jnp.float32)
    eo_ref[:, :h] = _elu(ep)
    eo_ref[:, h:] = _elu(pooled)
    if cast:
        rest[0][...] = mask_ref[...].astype(jnp.int8)


def _att_cat(mask, vecs, wa, wm, nea8, hp, cast=False):
    a, c, b, d = vecs
    r, cdim = mask.shape
    n = nea8.shape[0]
    h = wa.shape[1] - 1
    out_specs = [
        pl.BlockSpec((_BLK, h), lambda i: (i, 0)),
        pl.BlockSpec((_BLK, 2 * h), lambda i: (i, 0)),
    ]
    out_shape = [
        jax.ShapeDtypeStruct((r, h), jnp.float32),
        jax.ShapeDtypeStruct((r, 2 * h), jnp.float32),
    ]
    if cast:
        out_specs.append(pl.BlockSpec((_BLK, cdim), lambda i: (i, 0)))
        out_shape.append(jax.ShapeDtypeStruct((r, cdim), jnp.int8))
    return pl.pallas_call(
        functools.partial(_att_cat_body, h, cast),
        grid=(r // _BLK,),
        in_specs=[
            pl.BlockSpec((_BLK, cdim), lambda i: (i, 0)),
            pl.BlockSpec((1, _BLK), lambda i: (0, i)),
            pl.BlockSpec((1, _BLK), lambda i: (0, i)),
            pl.BlockSpec((1, cdim), lambda i: (0, 0)),
            pl.BlockSpec((1, cdim), lambda i: (0, 0)),
            pl.BlockSpec((cdim, h + 1), lambda i: (0, 0)),
            pl.BlockSpec((1, h), lambda i: (0, 0)),
            pl.BlockSpec((n, _BLK), lambda i: (0, i)),
            pl.BlockSpec((n, h), lambda i: (0, 0)),
        ],
        out_specs=tuple(out_specs),
        out_shape=tuple(out_shape),
    )(mask, a, c, b, d, wa, wm, nea8, hp)


def _cat_body(h, ne_ref, hp_ref, ep_ref, o_ref):
    pooled = jnp.dot(ne_ref[...].astype(jnp.float32), ep_ref[...],
                     preferred_element_type=jnp.float32)
    o_ref[:, :h] = _elu(hp_ref[...])
    o_ref[:, h:] = _elu(pooled)


def _cat(ne8, hp, ep):
    r, cdim = ne8.shape
    h = hp.shape[1]
    return pl.pallas_call(
        functools.partial(_cat_body, h),
        grid=(r // _BLK,),
        in_specs=[
            pl.BlockSpec((_BLK, cdim), lambda i: (i, 0)),
            pl.BlockSpec((_BLK, h), lambda i: (i, 0)),
            pl.BlockSpec((cdim, h), lambda i: (0, 0)),
        ],
        out_specs=pl.BlockSpec((_BLK, 2 * h), lambda i: (i, 0)),
        out_shape=jax.ShapeDtypeStruct((r, 2 * h), jnp.float32),
    )(ne8, hp, ep)


def _mlp_stats(pooled, w1_ref, b1_ref, w2_ref, b2_ref, z_ref, s_ref, ss_ref):
    t = jnp.maximum(
        jnp.dot(pooled, w1_ref[...], preferred_element_type=jnp.float32)
        + b1_ref[...], 0.0)
    z = jnp.dot(t, w2_ref[...], preferred_element_type=jnp.float32) + b2_ref[...]
    z_ref[...] = z

    @pl.when(pl.program_id(0) == 0)
    def _init():
        s_ref[...] = jnp.zeros_like(s_ref)
        ss_ref[...] = jnp.zeros_like(ss_ref)

    s_ref[...] += jnp.sum(z, axis=0, keepdims=True)
    ss_ref[...] += jnp.sum(z * z, axis=0, keepdims=True)


def _pool1_body(a_ref, h_ref, w1_ref, b1_ref, w2_ref, b2_ref,
                z_ref, s_ref, ss_ref):
    pooled = jnp.dot(a_ref[...].astype(jnp.float32), h_ref[...],
                     preferred_element_type=jnp.float32)
    _mlp_stats(pooled, w1_ref, b1_ref, w2_ref, b2_ref, z_ref, s_ref, ss_ref)


def _pool2_body(n, a_ref, zp_ref, sp_ref, ssp_ref, g_ref, bt_ref,
                w1_ref, b1_ref, w2_ref, b2_ref, z_ref, s_ref, ss_ref):
    h = _bn_relu_expr(zp_ref[...], sp_ref[...], ssp_ref[...],
                      g_ref[...], bt_ref[...], n)
    pooled = jnp.dot(a_ref[...].astype(jnp.float32), h,
                     preferred_element_type=jnp.float32)
    _mlp_stats(pooled, w1_ref, b1_ref, w2_ref, b2_ref, z_ref, s_ref, ss_ref)


def _pool_outs(r, k2):
    return (
        (
            pl.BlockSpec((_BLK, k2), lambda i: (i, 0)),
            pl.BlockSpec((1, k2), lambda i: (0, 0)),
            pl.BlockSpec((1, k2), lambda i: (0, 0)),
        ),
        (
            jax.ShapeDtypeStruct((r, k2), jnp.float32),
            jax.ShapeDtypeStruct((1, k2), jnp.float32),
            jax.ShapeDtypeStruct((1, k2), jnp.float32),
        ),
    )


def _pool1(a8, h, mp):
    r, cdim = a8.shape
    f = h.shape[1]
    k1 = mp['W1'].shape[1]
    k2 = mp['W2'].shape[1]
    out_specs, out_shape = _pool_outs(r, k2)
    return pl.pallas_call(
        _pool1_body,
        grid=(r // _BLK,),
        in_specs=[
            pl.BlockSpec((_BLK, cdim), lambda i: (i, 0)),
            pl.BlockSpec((cdim, f), lambda i: (0, 0)),
            pl.BlockSpec((f, k1), lambda i: (0, 0)),
            pl.BlockSpec((1, k1), lambda i: (0, 0)),
            pl.BlockSpec((k1, k2), lambda i: (0, 0)),
            pl.BlockSpec((1, k2), lambda i: (0, 0)),
        ],
        out_specs=out_specs,
        out_shape=out_shape,
    )(a8, h, mp['W1'], mp['b1'].reshape(1, -1), mp['W2'],
      mp['b2'].reshape(1, -1))


def _pool2(a8, zp, sp, ssp, bp, mp):
    r, cdim = a8.shape
    f = zp.shape[1]
    k1 = mp['W1'].shape[1]
    k2 = mp['W2'].shape[1]
    out_specs, out_shape = _pool_outs(r, k2)
    return pl.pallas_call(
        functools.partial(_pool2_body, float(cdim)),
        grid=(r // _BLK,),
        in_specs=[
            pl.BlockSpec((_BLK, cdim), lambda i: (i, 0)),
            pl.BlockSpec((cdim, f), lambda i: (0, 0)),
            pl.BlockSpec((1, f), lambda i: (0, 0)),
            pl.BlockSpec((1, f), lambda i: (0, 0)),
            pl.BlockSpec((1, f), lambda i: (0, 0)),
            pl.BlockSpec((1, f), lambda i: (0, 0)),
            pl.BlockSpec((f, k1), lambda i: (0, 0)),
            pl.BlockSpec((1, k1), lambda i: (0, 0)),
            pl.BlockSpec((k1, k2), lambda i: (0, 0)),
            pl.BlockSpec((1, k2), lambda i: (0, 0)),
        ],
        out_specs=out_specs,
        out_shape=out_shape,
    )(a8, zp, sp, ssp, bp['gamma'].reshape(1, -1), bp['beta'].reshape(1, -1),
      mp['W1'], mp['b1'].reshape(1, -1), mp['W2'], mp['b2'].reshape(1, -1))


def kernel(x, e_x, adj, e_adj, n_e_adj, params):
    nea8 = n_e_adj.astype(jnp.int8)

    xn, xe = x, e_x
    for li, lp in enumerate((params['in_att'], params['att0'])):
        (wan, wmn, av, cv, bv, dv,
         wae, wme, ae_, ce, be, de) = _proj(xn, xe, lp)
        if li == 0:
            hp, adj8 = _att(adj, (av, cv, bv, dv), wan, wmn, cast=True)
            ep, eo, eadj8 = _att_cat(e_adj, (ae_, ce, be, de), wae, wme,
                                     nea8, hp, cast=True)
        else:
            hp, = _att(adj8, (av, cv, bv, dv), wan, wmn)
            ep, eo = _att_cat(eadj8, (ae_, ce, be, de), wae, wme, nea8, hp)
        xo = _cat(nea8, hp, ep)
        xn, xe = xo, eo

    zn, sn, ssn = _pool1(adj8, xn, params['mlp0'])
    ze, se, sse = _pool1(eadj8, xe, params['mlp0'])
    zn, sn2, ssn2 = _pool2(adj8, zn, sn, ssn, params['bn0'], params['mlp1'])
    ze, se2, sse2 = _pool2(eadj8, ze, se, sse, params['bn0'], params['mlp1'])

    (wan, wmn, av, cv, bv, dv,
     wae, wme, ae_, ce, be, de) = _projbn(zn, sn2, ssn2, ze, se2, sse2,
                                          params['bn1'], params['out_att'])
    fx, = _att(adj8, (av, cv, bv, dv), wan, wmn, final=True)
    fe, = _att(eadj8, (ae_, ce, be, de), wae, wme, final=True)
    return fx, fe


# speedup vs baseline: 2.0963x; 1.0530x over previous
import functools

import jax
import jax.numpy as jnp
from jax.experimental import pallas as pl

_ALPHA = 0.2
_BLK = 512
_EPS = 1e-5


def _abcd(wh, an, h):
    f1 = jax.lax.dot_general(an[:h, :], wh, (((0,), (1,)), ((), ())),
                             preferred_element_type=jnp.float32)
    f2 = jax.lax.dot_general(an[h:, :], wh, (((0,), (1,)), ((), ())),
                             preferred_element_type=jnp.float32)
    mf2 = jnp.max(f2)
    t = f1 + mf2
    m = jnp.maximum(t, _ALPHA * t)
    a = jnp.exp(t - m)
    c = jnp.exp(_ALPHA * t - m)
    b = jnp.exp(f2 - mf2)
    d = jnp.exp(_ALPHA * (f2 - mf2))
    return a, c, b, d


def _elu(x):
    return jnp.where(x > 0, x, jnp.exp(x) - 1.0)


def _proj_side(x, wn, an, h, wa_ref, wm_ref, a_ref, c_ref, b_ref, d_ref):
    wh = jnp.dot(x, wn, preferred_element_type=jnp.float32)
    wa_ref[:, :h] = wh.astype(jnp.bfloat16)
    wa_ref[:, h:] = jnp.ones_like(wa_ref[:, h:])
    wm_ref[...] = jnp.mean(wh, axis=0, keepdims=True)
    a, c, b, d = _abcd(wh, an, h)
    a_ref[...] = a
    c_ref[...] = c
    b_ref[...] = b
    d_ref[...] = d


def _proj_body(hn, he, xn_ref, wn_ref, an_ref, xe_ref, we_ref, ae_ref,
               wan_ref, wmn_ref, an1_ref, cn_ref, bn1_ref, dn_ref,
               wae_ref, wme_ref, ae1_ref, ce_ref, be_ref, de_ref):
    _proj_side(xn_ref[...], wn_ref[...], an_ref[...], hn,
               wan_ref, wmn_ref, an1_ref, cn_ref, bn1_ref, dn_ref)
    _proj_side(xe_ref[...], we_ref[...], ae_ref[...], he,
               wae_ref, wme_ref, ae1_ref, ce_ref, be_ref, de_ref)


def _side_shapes(r, h):
    return (
        jax.ShapeDtypeStruct((r, h + 1), jnp.bfloat16),
        jax.ShapeDtypeStruct((1, h), jnp.float32),
        jax.ShapeDtypeStruct((1, r), jnp.float32),
        jax.ShapeDtypeStruct((1, r), jnp.float32),
        jax.ShapeDtypeStruct((1, r), jnp.float32),
        jax.ShapeDtypeStruct((1, r), jnp.float32),
    )


def _proj(xn, xe, p):
    rn = xn.shape[0]
    re = xe.shape[0]
    hn = p['Wn'].shape[1]
    he = p['We'].shape[1]
    return pl.pallas_call(
        functools.partial(_proj_body, hn, he),
        out_shape=_side_shapes(rn, hn) + _side_shapes(re, he),
    )(xn, p['Wn'], p['an'], xe, p['We'], p['ae'])


def _bn_relu_expr(z, s, ss, gamma, beta, n):
    mu = s / n
    var = ss / n - mu * mu
    return jnp.maximum((z - mu) / jnp.sqrt(var + _EPS) * gamma + beta, 0.0)


def _projbn_body(hn, he, nn, ne,
                 zn_ref, sn_ref, ssn_ref, ze_ref, se_ref, sse_ref,
                 g_ref, bt_ref, wn_ref, an_ref, we_ref, ae_ref,
                 wan_ref, wmn_ref, an1_ref, cn_ref, bn1_ref, dn_ref,
                 wae_ref, wme_ref, ae1_ref, ce_ref, be_ref, de_ref):
    xn = _bn_relu_expr(zn_ref[...], sn_ref[...], ssn_ref[...],
                       g_ref[...], bt_ref[...], nn)
    xe = _bn_relu_expr(ze_ref[...], se_ref[...], sse_ref[...],
                       g_ref[...], bt_ref[...], ne)
    _proj_side(xn, wn_ref[...], an_ref[...], hn,
               wan_ref, wmn_ref, an1_ref, cn_ref, bn1_ref, dn_ref)
    _proj_side(xe, we_ref[...], ae_ref[...], he,
               wae_ref, wme_ref, ae1_ref, ce_ref, be_ref, de_ref)


def _projbn(zn, sn, ssn, ze, se, sse, bp, p):
    rn = zn.shape[0]
    re = ze.shape[0]
    hn = p['Wn'].shape[1]
    he = p['We'].shape[1]
    return pl.pallas_call(
        functools.partial(_projbn_body, hn, he, float(rn), float(re)),
        out_shape=_side_shapes(rn, hn) + _side_shapes(re, he),
    )(zn, sn, ssn, ze, se, sse,
      bp['gamma'].reshape(1, -1), bp['beta'].reshape(1, -1),
      p['Wn'], p['an'], p['We'], p['ae'])


def _att_num(mask_ref, a_ref, c_ref, b_ref, d_ref):
    maskb = mask_ref[...].astype(jnp.bfloat16)
    at = jnp.transpose(a_ref[...]).astype(jnp.bfloat16)
    ct = jnp.transpose(c_ref[...]).astype(jnp.bfloat16)
    b = b_ref[...].astype(jnp.bfloat16)
    d = d_ref[...].astype(jnp.bfloat16)
    return maskb * jnp.maximum(at * b, ct * d)


def _att_out(p, w_ref, wm_ref, h):
    ha = jnp.dot(p, w_ref[...], preferred_element_type=jnp.float32)
    den = ha[:, h:h + 1]
    return jnp.where(den > 0, ha[:, :h] / den, wm_ref[...])


def _att_body(h, final, cast, mask_ref, a_ref, c_ref, b_ref, d_ref, w_ref,
              wm_ref, o_ref, *rest):
    out = _att_out(_att_num(mask_ref, a_ref, c_ref, b_ref, d_ref),
                   w_ref, wm_ref, h)
    if final:
        out = _elu(out)
        out = out - jnp.max(out, axis=1, keepdims=True)
        out = out - jnp.log(jnp.sum(jnp.exp(out), axis=1, keepdims=True))
    o_ref[...] = out
    if cast:
        rest[0][...] = mask_ref[...].astype(jnp.int8)


def _att(mask, vecs, wa, wm, final=False, cast=False):
    a, c, b, d = vecs
    r, cdim = mask.shape
    h = wa.shape[1] - 1
    out_specs = [pl.BlockSpec((_BLK, h), lambda i: (i, 0))]
    out_shape = [jax.ShapeDtypeStruct((r, h), jnp.float32)]
    if cast:
        out_specs.append(pl.BlockSpec((_BLK, cdim), lambda i: (i, 0)))
        out_shape.append(jax.ShapeDtypeStruct((r, cdim), jnp.int8))
    return pl.pallas_call(
        functools.partial(_att_body, h, final, cast),
        grid=(r // _BLK,),
        in_specs=[
            pl.BlockSpec((_BLK, cdim), lambda i: (i, 0)),
            pl.BlockSpec((1, _BLK), lambda i: (0, i)),
            pl.BlockSpec((1, _BLK), lambda i: (0, i)),
            pl.BlockSpec((1, cdim), lambda i: (0, 0)),
            pl.BlockSpec((1, cdim), lambda i: (0, 0)),
            pl.BlockSpec((cdim, h + 1), lambda i: (0, 0)),
            pl.BlockSpec((1, h), lambda i: (0, 0)),
        ],
        out_specs=tuple(out_specs),
        out_shape=tuple(out_shape),
    )(mask, a, c, b, d, wa, wm)


def _att_cat_body(h, cast, mask_ref, a_ref, c_ref, b_ref, d_ref, w_ref,
                  wm_ref, nec_ref, hp_ref, ep_ref, eo_ref, *rest):
    ep = _att_out(_att_num(mask_ref, a_ref, c_ref, b_ref, d_ref),
                  w_ref, wm_ref, h)
    ep_ref[...] = ep
    pooled = jax.lax.dot_general(
        nec_ref[...].astype(jnp.bfloat16), hp_ref[...].astype(jnp.bfloat16),
        (((0,), (0,)), ((), ())), preferred_element_type=jnp.float32)
    eo_ref[:, :h] = _elu(ep)
    eo_ref[:, h:] = _elu(pooled)
    if cast:
        rest[0][...] = mask_ref[...].astype(jnp.int8)


def _att_cat(mask, vecs, wa, wm, nea8, hp, cast=False):
    a, c, b, d = vecs
    r, cdim = mask.shape
    n = nea8.shape[0]
    h = wa.shape[1] - 1
    out_specs = [
        pl.BlockSpec((_BLK, h), lambda i: (i, 0)),
        pl.BlockSpec((_BLK, 2 * h), lambda i: (i, 0)),
    ]
    out_shape = [
        jax.ShapeDtypeStruct((r, h), jnp.float32),
        jax.ShapeDtypeStruct((r, 2 * h), jnp.float32),
    ]
    if cast:
        out_specs.append(pl.BlockSpec((_BLK, cdim), lambda i: (i, 0)))
        out_shape.append(jax.ShapeDtypeStruct((r, cdim), jnp.int8))
    return pl.pallas_call(
        functools.partial(_att_cat_body, h, cast),
        grid=(r // _BLK,),
        in_specs=[
            pl.BlockSpec((_BLK, cdim), lambda i: (i, 0)),
            pl.BlockSpec((1, _BLK), lambda i: (0, i)),
            pl.BlockSpec((1, _BLK), lambda i: (0, i)),
            pl.BlockSpec((1, cdim), lambda i: (0, 0)),
            pl.BlockSpec((1, cdim), lambda i: (0, 0)),
            pl.BlockSpec((cdim, h + 1), lambda i: (0, 0)),
            pl.BlockSpec((1, h), lambda i: (0, 0)),
            pl.BlockSpec((n, _BLK), lambda i: (0, i)),
            pl.BlockSpec((n, h), lambda i: (0, 0)),
        ],
        out_specs=tuple(out_specs),
        out_shape=tuple(out_shape),
    )(mask, a, c, b, d, wa, wm, nea8, hp)


def _cat_body(h, ne_ref, hp_ref, ep_ref, o_ref):
    pooled = jnp.dot(ne_ref[...].astype(jnp.bfloat16),
                     ep_ref[...].astype(jnp.bfloat16),
                     preferred_element_type=jnp.float32)
    o_ref[:, :h] = _elu(hp_ref[...])
    o_ref[:, h:] = _elu(pooled)


def _cat(ne8, hp, ep):
    r, cdim = ne8.shape
    h = hp.shape[1]
    return pl.pallas_call(
        functools.partial(_cat_body, h),
        grid=(r // _BLK,),
        in_specs=[
            pl.BlockSpec((_BLK, cdim), lambda i: (i, 0)),
            pl.BlockSpec((_BLK, h), lambda i: (i, 0)),
            pl.BlockSpec((cdim, h), lambda i: (0, 0)),
        ],
        out_specs=pl.BlockSpec((_BLK, 2 * h), lambda i: (i, 0)),
        out_shape=jax.ShapeDtypeStruct((r, 2 * h), jnp.float32),
    )(ne8, hp, ep)


def _mlp_stats(pooled, w1_ref, b1_ref, w2_ref, b2_ref, z_ref, s_ref, ss_ref):
    t = jnp.maximum(
        jnp.dot(pooled, w1_ref[...], preferred_element_type=jnp.float32)
        + b1_ref[...], 0.0)
    z = jnp.dot(t, w2_ref[...], preferred_element_type=jnp.float32) + b2_ref[...]
    z_ref[...] = z

    @pl.when(pl.program_id(0) == 0)
    def _init():
        s_ref[...] = jnp.zeros_like(s_ref)
        ss_ref[...] = jnp.zeros_like(ss_ref)

    s_ref[...] += jnp.sum(z, axis=0, keepdims=True)
    ss_ref[...] += jnp.sum(z * z, axis=0, keepdims=True)


def _pool1_body(a_ref, h_ref, w1_ref, b1_ref, w2_ref, b2_ref,
                z_ref, s_ref, ss_ref):
    pooled = jnp.dot(a_ref[...].astype(jnp.bfloat16),
                     h_ref[...].astype(jnp.bfloat16),
                     preferred_element_type=jnp.float32)
    _mlp_stats(pooled, w1_ref, b1_ref, w2_ref, b2_ref, z_ref, s_ref, ss_ref)


def _pool2_body(n, a_ref, zp_ref, sp_ref, ssp_ref, g_ref, bt_ref,
                w1_ref, b1_ref, w2_ref, b2_ref, z_ref, s_ref, ss_ref):
    h = _bn_relu_expr(zp_ref[...], sp_ref[...], ssp_ref[...],
                      g_ref[...], bt_ref[...], n)
    pooled = jnp.dot(a_ref[...].astype(jnp.bfloat16), h.astype(jnp.bfloat16),
                     preferred_element_type=jnp.float32)
    _mlp_stats(pooled, w1_ref, b1_ref, w2_ref, b2_ref, z_ref, s_ref, ss_ref)


def _pool_outs(r, k2):
    return (
        (
            pl.BlockSpec((_BLK, k2), lambda i: (i, 0)),
            pl.BlockSpec((1, k2), lambda i: (0, 0)),
            pl.BlockSpec((1, k2), lambda i: (0, 0)),
        ),
        (
            jax.ShapeDtypeStruct((r, k2), jnp.float32),
            jax.ShapeDtypeStruct((1, k2), jnp.float32),
            jax.ShapeDtypeStruct((1, k2), jnp.float32),
        ),
    )


def _pool1(a8, h, mp):
    r, cdim = a8.shape
    f = h.shape[1]
    k1 = mp['W1'].shape[1]
    k2 = mp['W2'].shape[1]
    out_specs, out_shape = _pool_outs(r, k2)
    return pl.pallas_call(
        _pool1_body,
        grid=(r // _BLK,),
        in_specs=[
            pl.BlockSpec((_BLK, cdim), lambda i: (i, 0)),
            pl.BlockSpec((cdim, f), lambda i: (0, 0)),
            pl.BlockSpec((f, k1), lambda i: (0, 0)),
            pl.BlockSpec((1, k1), lambda i: (0, 0)),
            pl.BlockSpec((k1, k2), lambda i: (0, 0)),
            pl.BlockSpec((1, k2), lambda i: (0, 0)),
        ],
        out_specs=out_specs,
        out_shape=out_shape,
    )(a8, h, mp['W1'], mp['b1'].reshape(1, -1), mp['W2'],
      mp['b2'].reshape(1, -1))


def _pool2(a8, zp, sp, ssp, bp, mp):
    r, cdim = a8.shape
    f = zp.shape[1]
    k1 = mp['W1'].shape[1]
    k2 = mp['W2'].shape[1]
    out_specs, out_shape = _pool_outs(r, k2)
    return pl.pallas_call(
        functools.partial(_pool2_body, float(cdim)),
        grid=(r // _BLK,),
        in_specs=[
            pl.BlockSpec((_BLK, cdim), lambda i: (i, 0)),
            pl.BlockSpec((cdim, f), lambda i: (0, 0)),
            pl.BlockSpec((1, f), lambda i: (0, 0)),
            pl.BlockSpec((1, f), lambda i: (0, 0)),
            pl.BlockSpec((1, f), lambda i: (0, 0)),
            pl.BlockSpec((1, f), lambda i: (0, 0)),
            pl.BlockSpec((f, k1), lambda i: (0, 0)),
            pl.BlockSpec((1, k1), lambda i: (0, 0)),
            pl.BlockSpec((k1, k2), lambda i: (0, 0)),
            pl.BlockSpec((1, k2), lambda i: (0, 0)),
        ],
        out_specs=out_specs,
        out_shape=out_shape,
    )(a8, zp, sp, ssp, bp['gamma'].reshape(1, -1), bp['beta'].reshape(1, -1),
      mp['W1'], mp['b1'].reshape(1, -1), mp['W2'], mp['b2'].reshape(1, -1))


def kernel(x, e_x, adj, e_adj, n_e_adj, params):
    nea8 = n_e_adj.astype(jnp.int8)

    xn, xe = x, e_x
    for li, lp in enumerate((params['in_att'], params['att0'])):
        (wan, wmn, av, cv, bv, dv,
         wae, wme, ae_, ce, be, de) = _proj(xn, xe, lp)
        if li == 0:
            hp, adj8 = _att(adj, (av, cv, bv, dv), wan, wmn, cast=True)
            ep, eo, eadj8 = _att_cat(e_adj, (ae_, ce, be, de), wae, wme,
                                     nea8, hp, cast=True)
        else:
            hp, = _att(adj8, (av, cv, bv, dv), wan, wmn)
            ep, eo = _att_cat(eadj8, (ae_, ce, be, de), wae, wme, nea8, hp)
        xo = _cat(nea8, hp, ep)
        xn, xe = xo, eo

    zn, sn, ssn = _pool1(adj8, xn, params['mlp0'])
    ze, se, sse = _pool1(eadj8, xe, params['mlp0'])
    zn, sn2, ssn2 = _pool2(adj8, zn, sn, ssn, params['bn0'], params['mlp1'])
    ze, se2, sse2 = _pool2(eadj8, ze, se, sse, params['bn0'], params['mlp1'])

    (wan, wmn, av, cv, bv, dv,
     wae, wme, ae_, ce, be, de) = _projbn(zn, sn2, ssn2, ze, se2, sse2,
                                          params['bn1'], params['out_att'])
    fx, = _att(adj8, (av, cv, bv, dv), wan, wmn, final=True)
    fe, = _att(eadj8, (ae_, ce, be, de), wae, wme, final=True)
    return fx, fe
